# baseline (speedup 1.0000x reference)
import functools

import jax
import jax.numpy as jnp
from jax import lax
from jax.experimental import pallas as pl
from jax.experimental.pallas import tpu as pltpu
from jax.experimental.pallas import tpu_sc as plsc

NC, NS = 2, 16
NW = NC * NS
B, L = 1024, 200
ROW = L * L
BPW = B // NW
P0 = 1016
NCELL = 1216
RU = 2
OFFS = tuple(range(0, 192, 16)) + (184,)


def _build_packed_table(boundaries):
    nb = boundaries.shape[0]
    p = jnp.arange(NCELL, dtype=jnp.int32)
    s = lax.bitcast_convert_type(p << 20, jnp.float32)
    s_next = lax.bitcast_convert_type((p + 1) << 20, jnp.float32)
    dlo = jnp.ceil(s)
    base = jnp.searchsorted(boundaries, dlo, side="left").astype(jnp.int32)
    cand = jnp.minimum(base, nb - 1)
    bcand = boundaries[cand]
    has_thr = (base < nb) & (bcand < s_next)
    thr = jnp.where(has_thr, bcand, 2.0 ** 24).astype(jnp.int32)
    return (thr << 6) | base


@functools.cache
def _make_sc_bias_kernel():
    mesh = plsc.VectorSubcoreMesh(
        core_axis_name="c", subcore_axis_name="s", num_cores=NC)

    @functools.partial(
        pl.kernel,
        out_type=jax.ShapeDtypeStruct((B, ROW), jnp.float32),
        mesh=mesh,
        compiler_params=pltpu.CompilerParams(needs_layout_passes=False),
        scratch_types=[
            pltpu.VMEM((BPW * L,), jnp.int32),
            pltpu.VMEM((BPW * L,), jnp.int32),
            pltpu.VMEM((NCELL,), jnp.int32),
            pltpu.VMEM((64,), jnp.float32),
            pltpu.VMEM((ROW,), jnp.float32),
            pltpu.VMEM((ROW,), jnp.float32),
            pltpu.SemaphoreType.DMA,
            pltpu.SemaphoreType.DMA,
        ],
    )
    def _sc_bias_kernel(tsq_hbm, tsk_hbm, packed_hbm, tb_hbm, out_hbm,
                        tsq_v, tsk_v, packed_v, tb_v, out_v0, out_v1,
                        sem0, sem1):
        wid = lax.axis_index("s") * NC + lax.axis_index("c")
        b0 = wid * BPW
        pltpu.sync_copy(tsq_hbm.at[pl.ds(b0 * L, BPW * L)], tsq_v)
        pltpu.sync_copy(tsk_hbm.at[pl.ds(b0 * L, BPW * L)], tsk_v)
        pltpu.sync_copy(packed_hbm, packed_v)
        pltpu.sync_copy(tb_hbm, tb_v)

        def compute_batch(bl, out_ref):
            kb = bl * L
            ks = [tsk_v[pl.ds(kb + off, 16)] for off in OFFS]

            def row_body(r, kcarry):
                qs = [plsc.load_gather(
                    tsq_v, [jnp.full((16,), kb + r * RU + rr, dtype=jnp.int32)])
                    for rr in range(RU)]
                work = [(rr, u) for rr in range(RU)
                        for u in range(len(OFFS))]
                ds = [jnp.maximum(jnp.abs(qs[rr] - kcarry[u]), 1)
                      for rr, u in work]
                cells = [lax.bitcast_convert_type(d.astype(jnp.float32),
                                                  jnp.int32) >> 20 for d in ds]
                pks = [plsc.load_gather(packed_v, [c]) for c in cells]
                buckets = [jnp.where(d > (pk >> 6), (pk & 63) + 1, pk & 63)
                           for d, pk in zip(ds, pks)]
                vals = [plsc.load_gather(tb_v, [b]) for b in buckets]
                for (rr, u), val in zip(work, vals):
                    out_ref[pl.ds((r * RU + rr) * L + OFFS[u], 16)] = val
                return kcarry

            lax.fori_loop(0, L // RU, row_body, tuple(ks))

        def pair_body(bp, carry):
            bl = 2 * bp

            @pl.when(bp >= 1)
            def _():
                pltpu.make_async_copy(out_v0, out_hbm.at[b0 + bl - 2],
                                      sem0).wait()

            compute_batch(bl, out_v0)
            pltpu.async_copy(out_v0, out_hbm.at[b0 + bl], sem0)

            @pl.when(bp >= 1)
            def _():
                pltpu.make_async_copy(out_v1, out_hbm.at[b0 + bl - 1],
                                      sem1).wait()

            compute_batch(bl + 1, out_v1)
            pltpu.async_copy(out_v1, out_hbm.at[b0 + bl + 1], sem1)
            return carry

        lax.fori_loop(0, BPW // 2, pair_body, 0)
        pltpu.make_async_copy(out_v0, out_hbm.at[b0 + BPW - 2], sem0).wait()
        pltpu.make_async_copy(out_v1, out_hbm.at[b0 + BPW - 1], sem1).wait()

    return _sc_bias_kernel


def kernel(ts_q, ts_k, time_bias, boundaries):
    assert ts_q.shape == (B, L) and ts_k.shape == (B, L)
    tsq = ts_q.astype(jnp.int32).reshape(B * L)
    tsk = ts_k.astype(jnp.int32).reshape(B * L)
    packed = _build_packed_table(boundaries)
    tb = time_bias[:, 0]
    out = _make_sc_bias_kernel()(tsq, tsk, packed, tb)
    return out.reshape(B, 1, L, L)


# speedup vs baseline: 1.0375x; 1.0375x over previous
import functools

import jax
import jax.numpy as jnp
from jax import lax
from jax.experimental import pallas as pl
from jax.experimental.pallas import tpu as pltpu
from jax.experimental.pallas import tpu_sc as plsc

NC, NS = 2, 16
NW = NC * NS
B, L = 1024, 200
ROW = L * L
BPW = B // NW
P0 = 1016
NCELL = 1216
RU = 2
OFFS = tuple(range(0, 192, 16)) + (184,)


def _build_packed_table(boundaries):
    nb = boundaries.shape[0]
    p = jnp.arange(NCELL, dtype=jnp.int32)
    s = lax.bitcast_convert_type(p << 20, jnp.float32)
    s_next = lax.bitcast_convert_type((p + 1) << 20, jnp.float32)
    dlo = jnp.ceil(s)
    base = jnp.searchsorted(boundaries, dlo, side="left").astype(jnp.int32)
    cand = jnp.minimum(base, nb - 1)
    bcand = boundaries[cand]
    has_thr = (base < nb) & (bcand < s_next)
    thr = jnp.where(has_thr, bcand, 2.0 ** 24).astype(jnp.int32)
    packed = (thr << 6) | base
    return packed.at[0].set(0)


@functools.cache
def _make_sc_bias_kernel():
    mesh = plsc.VectorSubcoreMesh(
        core_axis_name="c", subcore_axis_name="s", num_cores=NC)

    @functools.partial(
        pl.kernel,
        out_type=jax.ShapeDtypeStruct((B, ROW), jnp.float32),
        mesh=mesh,
        compiler_params=pltpu.CompilerParams(needs_layout_passes=False),
        scratch_types=[
            pltpu.VMEM((BPW * L,), jnp.int32),
            pltpu.VMEM((BPW * L,), jnp.int32),
            pltpu.VMEM((NCELL,), jnp.int32),
            pltpu.VMEM((64,), jnp.float32),
            pltpu.VMEM((ROW,), jnp.float32),
            pltpu.VMEM((ROW,), jnp.float32),
            pltpu.SemaphoreType.DMA,
            pltpu.SemaphoreType.DMA,
        ],
    )
    def _sc_bias_kernel(tsq_hbm, tsk_hbm, packed_hbm, tb_hbm, out_hbm,
                        tsq_v, tsk_v, packed_v, tb_v, out_v0, out_v1,
                        sem0, sem1):
        wid = lax.axis_index("s") * NC + lax.axis_index("c")
        b0 = wid * BPW
        pltpu.sync_copy(tsq_hbm.at[pl.ds(b0 * L, BPW * L)], tsq_v)
        pltpu.sync_copy(tsk_hbm.at[pl.ds(b0 * L, BPW * L)], tsk_v)
        pltpu.sync_copy(packed_hbm, packed_v)
        pltpu.sync_copy(tb_hbm, tb_v)

        def compute_batch(bl, out_ref):
            kb = bl * L

            def row_body(r, carry2):
                ks = [tsk_v[pl.ds(kb + off, 16)] for off in OFFS]
                qs = [plsc.load_gather(
                    tsq_v, [jnp.full((16,), kb + r * RU + rr, dtype=jnp.int32)])
                    for rr in range(RU)]
                work = [(rr, u) for rr in range(RU)
                        for u in range(len(OFFS))]
                ds = [jnp.abs(qs[rr] - ks[u]) for rr, u in work]
                cells = [lax.bitcast_convert_type(d.astype(jnp.float32),
                                                  jnp.int32) >> 20 for d in ds]
                pks = [plsc.load_gather(packed_v, [c]) for c in cells]
                buckets = [jnp.where(d > (pk >> 6), (pk & 63) + 1, pk & 63)
                           for d, pk in zip(ds, pks)]
                vals = [plsc.load_gather(tb_v, [b]) for b in buckets]
                for (rr, u), val in zip(work, vals):
                    out_ref[pl.ds((r * RU + rr) * L + OFFS[u], 16)] = val
                return carry2

            lax.fori_loop(0, L // RU, row_body, 0)

        def pair_body(bp, carry):
            bl = 2 * bp

            @pl.when(bp >= 1)
            def _():
                pltpu.make_async_copy(out_v0, out_hbm.at[b0 + bl - 2],
                                      sem0).wait()

            compute_batch(bl, out_v0)
            pltpu.async_copy(out_v0, out_hbm.at[b0 + bl], sem0)

            @pl.when(bp >= 1)
            def _():
                pltpu.make_async_copy(out_v1, out_hbm.at[b0 + bl - 1],
                                      sem1).wait()

            compute_batch(bl + 1, out_v1)
            pltpu.async_copy(out_v1, out_hbm.at[b0 + bl + 1], sem1)
            return carry

        lax.fori_loop(0, BPW // 2, pair_body, 0)
        pltpu.make_async_copy(out_v0, out_hbm.at[b0 + BPW - 2], sem0).wait()
        pltpu.make_async_copy(out_v1, out_hbm.at[b0 + BPW - 1], sem1).wait()

    return _sc_bias_kernel


def kernel(ts_q, ts_k, time_bias, boundaries):
    assert ts_q.shape == (B, L) and ts_k.shape == (B, L)
    tsq = ts_q.astype(jnp.int32).reshape(B * L)
    tsk = ts_k.astype(jnp.int32).reshape(B * L)
    packed = _build_packed_table(boundaries)
    tb = time_bias[:, 0]
    out = _make_sc_bias_kernel()(tsq, tsk, packed, tb)
    return out.reshape(B, 1, L, L)
